# baseline (speedup 1.0000x reference)
import functools

import jax
import jax.numpy as jnp
from jax import lax
from jax.experimental import pallas as pl
from jax.experimental.pallas import tpu as pltpu
from jax.experimental.pallas import tpu_sc as plsc

_VOCAB = 1000000
_EMBED_DIM = 64
_SCALE = 8.0

_NW = 32
_BLK = 128
_NFULL = _VOCAB // _BLK
_BLK_IT = _NFULL // _NW + 1
_CHUNK = 128
_GBUF = 2


def _prep_body(tabt_hbm, rem_hbm, out_hbm, bufin, bufout, rsems, wsems,
               *, nc):
  wid = lax.axis_index("s") * nc + lax.axis_index("c")

  iota = lax.iota(jnp.int32, 16)
  rowb = lax.shift_right_logical(iota, 1)
  colb = (iota & 1) * _EMBED_DIM

  def transpose_block(bi, kmax):
    @pl.loop(0, _EMBED_DIM)
    def _feat(c):
      cols = colb + c
      for k in range(kmax):
        v = bufin[bi][c, pl.ds(k * 16, 16)] * _SCALE
        plsc.store_scatter(bufout[bi], [rowb + 8 * k, cols], v)

  def blk_of(i):
    return wid + _NW * i

  for b in range(2):
    @pl.when(blk_of(b) < _NFULL)
    def _():
      off = pl.multiple_of(blk_of(b) * _BLK, _BLK)
      pltpu.async_copy(tabt_hbm.at[:, pl.ds(off, _BLK)], bufin[b], rsems[b])

  @pl.loop(0, _BLK_IT)
  def _blocks(i):
    b = lax.rem(i, 2)
    for bi in range(2):
      @pl.when((b == bi) & (blk_of(i) < _NFULL))
      def _():
        blk = blk_of(i)
        off = pl.multiple_of(blk * _BLK, _BLK)
        pltpu.make_async_copy(tabt_hbm.at[:, pl.ds(off, _BLK)], bufin[bi],
                              rsems[bi]).wait()

        @pl.when(i >= 2)
        def _():
          pltpu.make_async_copy(
              bufout[bi],
              out_hbm.at[pl.ds(0, _BLK // 2)], wsems[bi]).wait()

        transpose_block(bi, _BLK // 16)

        @pl.when(blk_of(i + 2) < _NFULL)
        def _():
          off2 = pl.multiple_of(blk_of(i + 2) * _BLK, _BLK)
          pltpu.async_copy(tabt_hbm.at[:, pl.ds(off2, _BLK)], bufin[bi],
                           rsems[bi])

        orow = pl.multiple_of(blk * (_BLK // 2), 8)
        pltpu.async_copy(bufout[bi],
                         out_hbm.at[pl.ds(orow, _BLK // 2)], wsems[bi])

  for bi in range(2):
    i_last = _BLK_IT - 1 if (_BLK_IT - 1) % 2 == bi else _BLK_IT - 2
    @pl.when(blk_of(i_last) < _NFULL)
    def _():
      pltpu.make_async_copy(bufout[bi],
                            out_hbm.at[pl.ds(0, _BLK // 2)],
                            wsems[bi]).wait()

  @pl.when(wid == 0)
  def _():
    pltpu.sync_copy(rem_hbm, bufin[0].at[pl.ds(0, 32), :])

    @pl.loop(0, 32)
    def _rem(r):
      for k in range(_BLK // 16):
        bufout[0][r, pl.ds(k * 16, 16)] = (
            bufin[0][r, pl.ds(k * 16, 16)] * _SCALE)

    pltpu.sync_copy(bufout[0].at[pl.ds(0, 32), :],
                    out_hbm.at[pl.ds(_NFULL * (_BLK // 2), 32)])


def _gather_body(tab_hbm, idx_hbm, out_hbm, idx_v, rows_v, gsems, wsems,
                 *, b_per_w, nc):
  wid = lax.axis_index("s") * nc + lax.axis_index("c")
  base = wid * b_per_w
  n_chunks = b_per_w // _CHUNK

  pltpu.sync_copy(idx_hbm.at[pl.ds(wid * n_chunks, n_chunks)], idx_v)

  def idx_slice(c):
    return idx_v.at[c]

  for b in range(_GBUF):
    pltpu.async_copy(tab_hbm.at[idx_slice(b)], rows_v[b], gsems[b])

  @pl.loop(0, n_chunks // _GBUF)
  def _pipeline(i):
    for b in range(_GBUF):
      c = i * _GBUF + b
      pltpu.make_async_copy(tab_hbm.at[idx_slice(c)], rows_v[b],
                            gsems[b]).wait()
      dst = out_hbm.at[pl.ds(base + c * _CHUNK, _CHUNK),
                       pl.ds(0, _EMBED_DIM)]
      pltpu.async_copy(rows_v[b], dst, wsems[b])

      @pl.when(c + _GBUF < n_chunks)
      def _():
        pltpu.make_async_copy(rows_v[b], dst, wsems[b]).wait()
        pltpu.async_copy(tab_hbm.at[idx_slice(c + _GBUF)], rows_v[b],
                         gsems[b])

  for b in range(_GBUF):
    pltpu.make_async_copy(
        rows_v[b],
        out_hbm.at[pl.ds(base, _CHUNK), pl.ds(0, _EMBED_DIM)],
        wsems[b]).wait()


@jax.jit
def _embed(x_flat, table):
  info = plsc.get_sparse_core_info()
  nc, ns = info.num_cores, info.num_subcores
  assert nc * ns == _NW
  n = x_flat.shape[0]
  b_per_w = n // _NW

  mesh = plsc.VectorSubcoreMesh(
      core_axis_name="c", subcore_axis_name="s",
      num_cores=nc, num_subcores=ns)

  prep = pl.kernel(
      functools.partial(_prep_body, nc=nc),
      out_type=jax.ShapeDtypeStruct((_VOCAB // 2, 2 * _EMBED_DIM),
                                    jnp.float32),
      mesh=mesh,
      scratch_types=dict(
          bufin=[pltpu.VMEM((_EMBED_DIM, _BLK), jnp.float32)
                 for _ in range(2)],
          bufout=[pltpu.VMEM((_BLK // 2, _BLK), jnp.float32)
                  for _ in range(2)],
          rsems=[pltpu.SemaphoreType.DMA for _ in range(2)],
          wsems=[pltpu.SemaphoreType.DMA for _ in range(2)],
      ),
      compiler_params=pltpu.CompilerParams(use_tc_tiling_on_sc=True,
                                           needs_layout_passes=False),
  )
  rem2 = table[_NFULL * _BLK:].reshape(32, 2 * _EMBED_DIM)
  scaled2 = prep(table.T, rem2)
  scaled = scaled2.reshape(_VOCAB, _EMBED_DIM)

  gather = pl.kernel(
      functools.partial(_gather_body, b_per_w=b_per_w, nc=nc),
      out_type=jax.ShapeDtypeStruct((n, 2 * _EMBED_DIM), jnp.float32),
      mesh=mesh,
      scratch_types=dict(
          idx_v=pltpu.VMEM((b_per_w // _CHUNK, _CHUNK), jnp.int32),
          rows_v=[pltpu.VMEM((_CHUNK, _EMBED_DIM), jnp.float32)
                  for _ in range(_GBUF)],
          gsems=[pltpu.SemaphoreType.DMA for _ in range(_GBUF)],
          wsems=[pltpu.SemaphoreType.DMA for _ in range(_GBUF)],
      ),
      compiler_params=pltpu.CompilerParams(use_tc_tiling_on_sc=False,
                                           needs_layout_passes=False),
  )
  return gather(scaled, x_flat.reshape(-1, _CHUNK))


def kernel(x, table):
  batch, hist = x.shape
  out128 = _embed(x.reshape(-1).astype(jnp.int32), table)
  return out128.reshape(batch, hist, 2 * _EMBED_DIM)[:, :, :_EMBED_DIM]


# speedup vs baseline: 1.4173x; 1.4173x over previous
import functools

import jax
import jax.numpy as jnp
from jax import lax
from jax.experimental import pallas as pl
from jax.experimental.pallas import tpu as pltpu
from jax.experimental.pallas import tpu_sc as plsc

_VOCAB = 1000000
_EMBED_DIM = 64
_SCALE = 8.0

_NW = 32
_BLK = 128
_NFULL = _VOCAB // _BLK
_BLK_IT = _NFULL // _NW + 1
_CHUNK = 128
_GBUF = 2


def _prep_body(tabt_hbm, rem_hbm, out_hbm, bufin, bufout, rsems, wsems,
               *, nc):
  wid = lax.axis_index("s") * nc + lax.axis_index("c")

  iota = lax.iota(jnp.int32, 16)
  rowb = lax.shift_right_logical(iota, 1)
  colb = (iota & 1) * _EMBED_DIM

  def transpose_block(bi, kmax):
    @plsc.parallel_loop(0, _EMBED_DIM, unroll=4)
    def _feat(c):
      cols = colb + c
      for k in range(kmax):
        v = bufin[bi][c, pl.ds(k * 16, 16)] * _SCALE
        plsc.store_scatter(bufout[bi], [rowb + 8 * k, cols], v)

  def blk_of(i):
    return wid + _NW * i

  for b in range(2):
    @pl.when(blk_of(b) < _NFULL)
    def _():
      off = pl.multiple_of(blk_of(b) * _BLK, _BLK)
      pltpu.async_copy(tabt_hbm.at[:, pl.ds(off, _BLK)], bufin[b], rsems[b])

  @pl.loop(0, _BLK_IT)
  def _blocks(i):
    b = lax.rem(i, 2)
    for bi in range(2):
      @pl.when((b == bi) & (blk_of(i) < _NFULL))
      def _():
        blk = blk_of(i)
        off = pl.multiple_of(blk * _BLK, _BLK)
        pltpu.make_async_copy(tabt_hbm.at[:, pl.ds(off, _BLK)], bufin[bi],
                              rsems[bi]).wait()

        @pl.when(i >= 2)
        def _():
          pltpu.make_async_copy(
              bufout[bi],
              out_hbm.at[pl.ds(0, _BLK // 2)], wsems[bi]).wait()

        transpose_block(bi, _BLK // 16)

        @pl.when(blk_of(i + 2) < _NFULL)
        def _():
          off2 = pl.multiple_of(blk_of(i + 2) * _BLK, _BLK)
          pltpu.async_copy(tabt_hbm.at[:, pl.ds(off2, _BLK)], bufin[bi],
                           rsems[bi])

        orow = pl.multiple_of(blk * (_BLK // 2), 8)
        pltpu.async_copy(bufout[bi],
                         out_hbm.at[pl.ds(orow, _BLK // 2)], wsems[bi])

  for bi in range(2):
    i_last = _BLK_IT - 1 if (_BLK_IT - 1) % 2 == bi else _BLK_IT - 2
    @pl.when(blk_of(i_last) < _NFULL)
    def _():
      pltpu.make_async_copy(bufout[bi],
                            out_hbm.at[pl.ds(0, _BLK // 2)],
                            wsems[bi]).wait()

  @pl.when(wid == 0)
  def _():
    pltpu.sync_copy(rem_hbm, bufin[0].at[pl.ds(0, 32), :])

    @pl.loop(0, 32)
    def _rem(r):
      for k in range(_BLK // 16):
        bufout[0][r, pl.ds(k * 16, 16)] = (
            bufin[0][r, pl.ds(k * 16, 16)] * _SCALE)

    pltpu.sync_copy(bufout[0].at[pl.ds(0, 32), :],
                    out_hbm.at[pl.ds(_NFULL * (_BLK // 2), 32)])


def _gather_body(tab_hbm, idx_hbm, out_hbm, idx_v, rows_v, gsems, wsems,
                 *, b_per_w, nc):
  wid = lax.axis_index("s") * nc + lax.axis_index("c")
  base = wid * b_per_w
  n_chunks = b_per_w // _CHUNK

  pltpu.sync_copy(idx_hbm.at[pl.ds(wid * n_chunks, n_chunks)], idx_v)

  def idx_slice(c):
    return idx_v.at[c]

  for b in range(_GBUF):
    pltpu.async_copy(tab_hbm.at[idx_slice(b)], rows_v[b], gsems[b])

  @pl.loop(0, n_chunks // _GBUF)
  def _pipeline(i):
    for b in range(_GBUF):
      c = i * _GBUF + b
      pltpu.make_async_copy(tab_hbm.at[idx_slice(c)], rows_v[b],
                            gsems[b]).wait()
      dst = out_hbm.at[pl.ds(base + c * _CHUNK, _CHUNK),
                       pl.ds(0, _EMBED_DIM)]
      pltpu.async_copy(rows_v[b], dst, wsems[b])

      @pl.when(c + _GBUF < n_chunks)
      def _():
        pltpu.make_async_copy(rows_v[b], dst, wsems[b]).wait()
        pltpu.async_copy(tab_hbm.at[idx_slice(c + _GBUF)], rows_v[b],
                         gsems[b])

  for b in range(_GBUF):
    pltpu.make_async_copy(
        rows_v[b],
        out_hbm.at[pl.ds(base, _CHUNK), pl.ds(0, _EMBED_DIM)],
        wsems[b]).wait()


@jax.jit
def _embed(x_flat, table):
  info = plsc.get_sparse_core_info()
  nc, ns = info.num_cores, info.num_subcores
  assert nc * ns == _NW
  n = x_flat.shape[0]
  b_per_w = n // _NW

  mesh = plsc.VectorSubcoreMesh(
      core_axis_name="c", subcore_axis_name="s",
      num_cores=nc, num_subcores=ns)

  prep = pl.kernel(
      functools.partial(_prep_body, nc=nc),
      out_type=jax.ShapeDtypeStruct((_VOCAB // 2, 2 * _EMBED_DIM),
                                    jnp.float32),
      mesh=mesh,
      scratch_types=dict(
          bufin=[pltpu.VMEM((_EMBED_DIM, _BLK), jnp.float32)
                 for _ in range(2)],
          bufout=[pltpu.VMEM((_BLK // 2, _BLK), jnp.float32)
                  for _ in range(2)],
          rsems=[pltpu.SemaphoreType.DMA for _ in range(2)],
          wsems=[pltpu.SemaphoreType.DMA for _ in range(2)],
      ),
      compiler_params=pltpu.CompilerParams(use_tc_tiling_on_sc=True,
                                           needs_layout_passes=False),
  )
  rem2 = table[_NFULL * _BLK:].reshape(32, 2 * _EMBED_DIM)
  scaled2 = prep(table.T, rem2)
  scaled = scaled2.reshape(_VOCAB, _EMBED_DIM)

  gather = pl.kernel(
      functools.partial(_gather_body, b_per_w=b_per_w, nc=nc),
      out_type=jax.ShapeDtypeStruct((n, 2 * _EMBED_DIM), jnp.float32),
      mesh=mesh,
      scratch_types=dict(
          idx_v=pltpu.VMEM((b_per_w // _CHUNK, _CHUNK), jnp.int32),
          rows_v=[pltpu.VMEM((_CHUNK, _EMBED_DIM), jnp.float32)
                  for _ in range(_GBUF)],
          gsems=[pltpu.SemaphoreType.DMA for _ in range(_GBUF)],
          wsems=[pltpu.SemaphoreType.DMA for _ in range(_GBUF)],
      ),
      compiler_params=pltpu.CompilerParams(use_tc_tiling_on_sc=False,
                                           needs_layout_passes=False),
  )
  return gather(scaled, x_flat.reshape(-1, _CHUNK))


def kernel(x, table):
  batch, hist = x.shape
  out128 = _embed(x.reshape(-1).astype(jnp.int32), table)
  return out128.reshape(batch, hist, 2 * _EMBED_DIM)[:, :, :_EMBED_DIM]


# speedup vs baseline: 1.8079x; 1.2756x over previous
import functools

import jax
import jax.numpy as jnp
from jax import lax
from jax.experimental import pallas as pl
from jax.experimental.pallas import tpu as pltpu
from jax.experimental.pallas import tpu_sc as plsc

_VOCAB = 1000000
_EMBED_DIM = 64
_SCALE = 8.0

_NW = 32
_BLK = 256
_NFULL = _VOCAB // _BLK
_BLK_IT = _NFULL // _NW + 1
_CHUNK = 128
_GBUF = 2


def _prep_body(tab_hbm, rem_hbm, out_hbm, bufin, bufout, rsems, wsems,
               *, nc):
  wid = lax.axis_index("s") * nc + lax.axis_index("c")

  def repack_block(bi, rows):
    @plsc.parallel_loop(0, rows, unroll=8)
    def _row(r):
      r2 = lax.shift_right_logical(r, 1)
      half = (r & 1) * _EMBED_DIM
      for k in range(_EMBED_DIM // 16):
        bufout[bi][r2, pl.ds(half + k * 16, 16)] = (
            bufin[bi][r, pl.ds(k * 16, 16)] * _SCALE)

  def blk_of(i):
    return wid + _NW * i

  for b in range(2):
    @pl.when(blk_of(b) < _NFULL)
    def _():
      off = pl.multiple_of(blk_of(b) * _BLK, _BLK)
      pltpu.async_copy(tab_hbm.at[pl.ds(off, _BLK), :], bufin[b], rsems[b])

  @pl.loop(0, _BLK_IT)
  def _blocks(i):
    b = lax.rem(i, 2)
    for bi in range(2):
      @pl.when((b == bi) & (blk_of(i) < _NFULL))
      def _():
        blk = blk_of(i)
        off = pl.multiple_of(blk * _BLK, _BLK)
        pltpu.make_async_copy(tab_hbm.at[pl.ds(off, _BLK), :], bufin[bi],
                              rsems[bi]).wait()

        @pl.when(i >= 2)
        def _():
          pltpu.make_async_copy(
              bufout[bi],
              out_hbm.at[pl.ds(0, _BLK // 2)], wsems[bi]).wait()

        repack_block(bi, _BLK)

        @pl.when(blk_of(i + 2) < _NFULL)
        def _():
          off2 = pl.multiple_of(blk_of(i + 2) * _BLK, _BLK)
          pltpu.async_copy(tab_hbm.at[pl.ds(off2, _BLK), :], bufin[bi],
                           rsems[bi])

        orow = pl.multiple_of(blk * (_BLK // 2), 8)
        pltpu.async_copy(bufout[bi],
                         out_hbm.at[pl.ds(orow, _BLK // 2)], wsems[bi])

  for bi in range(2):
    i_last = _BLK_IT - 1 if (_BLK_IT - 1) % 2 == bi else _BLK_IT - 2
    @pl.when(blk_of(i_last) < _NFULL)
    def _():
      pltpu.make_async_copy(bufout[bi],
                            out_hbm.at[pl.ds(0, _BLK // 2)],
                            wsems[bi]).wait()

  @pl.when(wid == 0)
  def _():
    pltpu.sync_copy(rem_hbm, bufout[0].at[pl.ds(0, 32), :])

    @pl.loop(0, 32)
    def _rem(r):
      for k in range(2 * _EMBED_DIM // 16):
        bufout[0][r, pl.ds(k * 16, 16)] = (
            bufout[0][r, pl.ds(k * 16, 16)] * _SCALE)

    pltpu.sync_copy(bufout[0].at[pl.ds(0, 32), :],
                    out_hbm.at[pl.ds(_NFULL * (_BLK // 2), 32)])


def _gather_body(tab_hbm, idx_hbm, out_hbm, idx_v, rows_v, gsems, wsems,
                 *, b_per_w, nc):
  wid = lax.axis_index("s") * nc + lax.axis_index("c")
  base = wid * b_per_w
  n_chunks = b_per_w // _CHUNK

  pltpu.sync_copy(idx_hbm.at[pl.ds(wid * n_chunks, n_chunks)], idx_v)

  def idx_slice(c):
    return idx_v.at[c]

  for b in range(_GBUF):
    pltpu.async_copy(tab_hbm.at[idx_slice(b)], rows_v[b], gsems[b])

  @pl.loop(0, n_chunks // _GBUF)
  def _pipeline(i):
    for b in range(_GBUF):
      c = i * _GBUF + b
      pltpu.make_async_copy(tab_hbm.at[idx_slice(c)], rows_v[b],
                            gsems[b]).wait()
      dst = out_hbm.at[pl.ds(base + c * _CHUNK, _CHUNK),
                       pl.ds(0, _EMBED_DIM)]
      pltpu.async_copy(rows_v[b], dst, wsems[b])

      @pl.when(c + _GBUF < n_chunks)
      def _():
        pltpu.make_async_copy(rows_v[b], dst, wsems[b]).wait()
        pltpu.async_copy(tab_hbm.at[idx_slice(c + _GBUF)], rows_v[b],
                         gsems[b])

  for b in range(_GBUF):
    pltpu.make_async_copy(
        rows_v[b],
        out_hbm.at[pl.ds(base, _CHUNK), pl.ds(0, _EMBED_DIM)],
        wsems[b]).wait()


@jax.jit
def _embed(x_flat, table):
  info = plsc.get_sparse_core_info()
  nc, ns = info.num_cores, info.num_subcores
  assert nc * ns == _NW
  n = x_flat.shape[0]
  b_per_w = n // _NW

  mesh = plsc.VectorSubcoreMesh(
      core_axis_name="c", subcore_axis_name="s",
      num_cores=nc, num_subcores=ns)

  prep = pl.kernel(
      functools.partial(_prep_body, nc=nc),
      out_type=jax.ShapeDtypeStruct((_VOCAB // 2, 2 * _EMBED_DIM),
                                    jnp.float32),
      mesh=mesh,
      scratch_types=dict(
          bufin=[pltpu.VMEM((_BLK, _EMBED_DIM), jnp.float32)
                 for _ in range(2)],
          bufout=[pltpu.VMEM((_BLK // 2, 2 * _EMBED_DIM), jnp.float32)
                  for _ in range(2)],
          rsems=[pltpu.SemaphoreType.DMA for _ in range(2)],
          wsems=[pltpu.SemaphoreType.DMA for _ in range(2)],
      ),
      compiler_params=pltpu.CompilerParams(use_tc_tiling_on_sc=True,
                                           needs_layout_passes=False),
  )
  rem2 = table[_NFULL * _BLK:].reshape(32, 2 * _EMBED_DIM)
  scaled2 = prep(table, rem2)
  scaled = scaled2.reshape(_VOCAB, _EMBED_DIM)

  gather = pl.kernel(
      functools.partial(_gather_body, b_per_w=b_per_w, nc=nc),
      out_type=jax.ShapeDtypeStruct((n, 2 * _EMBED_DIM), jnp.float32),
      mesh=mesh,
      scratch_types=dict(
          idx_v=pltpu.VMEM((b_per_w // _CHUNK, _CHUNK), jnp.int32),
          rows_v=[pltpu.VMEM((_CHUNK, _EMBED_DIM), jnp.float32)
                  for _ in range(_GBUF)],
          gsems=[pltpu.SemaphoreType.DMA for _ in range(_GBUF)],
          wsems=[pltpu.SemaphoreType.DMA for _ in range(_GBUF)],
      ),
      compiler_params=pltpu.CompilerParams(use_tc_tiling_on_sc=False,
                                           needs_layout_passes=False),
  )
  return gather(scaled, x_flat.reshape(-1, _CHUNK))


def kernel(x, table):
  batch, hist = x.shape
  out128 = _embed(x.reshape(-1).astype(jnp.int32), table)
  return out128.reshape(batch, hist, 2 * _EMBED_DIM)[:, :, :_EMBED_DIM]
